# baseline (speedup 1.0000x reference)
import functools

import jax
import jax.numpy as jnp
from jax import lax
from jax.experimental import pallas as pl
from jax.experimental.pallas import tpu as pltpu
from jax.experimental.pallas import tpu_sc as plsc

NUM_CORES = 2
NUM_SUBCORES = 16
NUM_WORKERS = NUM_CORES * NUM_SUBCORES
LANES = 128
PACK = 4
D_MODEL = 32
IDX_W = 128
STREAMS_PER_CHUNK = 8
CHUNK = IDX_W * STREAMS_PER_CHUNK
GROUPS = IDX_W // 16


def _make_gather(total):
    assert total % (NUM_WORKERS * CHUNK) == 0
    per_worker = total // NUM_WORKERS
    n_chunks = per_worker // CHUNK
    idx_rows_per_worker = per_worker // IDX_W

    mesh = plsc.VectorSubcoreMesh(core_axis_name="c", subcore_axis_name="s")

    @functools.partial(
        pl.kernel,
        out_type=jax.ShapeDtypeStruct((total * D_MODEL,), jnp.float32),
        mesh=mesh,
        compiler_params=pltpu.CompilerParams(needs_layout_passes=False),
        scratch_types=[
            pltpu.VMEM((STREAMS_PER_CHUNK, IDX_W), jnp.int32),
            pltpu.VMEM((STREAMS_PER_CHUNK, IDX_W), jnp.int32),
            pltpu.VMEM((IDX_W, LANES), jnp.float32),
            pltpu.VMEM((IDX_W, LANES), jnp.float32),
            pltpu.VMEM((CHUNK * D_MODEL,), jnp.float32),
            pltpu.SemaphoreType.DMA,
            pltpu.SemaphoreType.DMA,
        ],
    )
    def gather_kernel(idx_hbm, table_hbm, out_hbm, idx_v, q_v, wide0, wide1,
                      comp_v, sem0, sem1):
        wid = lax.axis_index("s") * NUM_CORES + lax.axis_index("c")
        iota = lax.iota(jnp.int32, 16)

        def fire(j, w, sem):
            pltpu.async_copy(table_hbm.at[q_v.at[j]], w, sem)

        def wait(j, w, sem):
            pltpu.make_async_copy(table_hbm.at[q_v.at[j]], w, sem).wait()

        def extract(j, w):
            base_flat = j * (IDX_W * D_MODEL)

            def g_body(g, carry):
                idx16 = idx_v[j, pl.ds(g * 16, 16)]
                rb = (idx16 & 3) * D_MODEL
                rowv = g * 16 + iota
                dst0 = base_flat + g * (16 * D_MODEL) + iota * D_MODEL
                for c in range(D_MODEL):
                    val = plsc.load_gather(w, [rowv, rb + c])
                    plsc.store_scatter(comp_v, [dst0 + c], val)
                return carry

            lax.fori_loop(0, GROUPS, g_body, 0)

        def chunk_body(i, carry):
            row0 = wid * idx_rows_per_worker + i * STREAMS_PER_CHUNK
            pltpu.sync_copy(
                idx_hbm.at[pl.ds(row0, STREAMS_PER_CHUNK)], idx_v
            )
            for j in range(STREAMS_PER_CHUNK):
                for k in range(GROUPS):
                    sl = pl.ds(k * 16, 16)
                    q_v[j, sl] = idx_v[j, sl] >> 2
            fire(0, wide0, sem0)
            fire(1, wide1, sem1)
            for j in range(STREAMS_PER_CHUNK):
                w = wide0 if j % 2 == 0 else wide1
                sem = sem0 if j % 2 == 0 else sem1
                wait(j, w, sem)
                extract(j, w)
                if j + 2 < STREAMS_PER_CHUNK:
                    fire(j + 2, w, sem)
            out_base = (wid * per_worker + i * CHUNK) * D_MODEL
            pltpu.sync_copy(
                comp_v, out_hbm.at[pl.ds(out_base, CHUNK * D_MODEL)]
            )
            return carry

        lax.fori_loop(0, n_chunks, chunk_body, 0)

    return gather_kernel


def kernel(inputs, embedding_matrix):
    batch, hist = inputs.shape
    total = batch * hist
    vocab, d_model = embedding_matrix.shape
    idx2d = inputs.reshape(total // IDX_W, IDX_W).astype(jnp.int32)
    table128 = embedding_matrix.reshape(vocab * d_model // LANES, LANES)
    out = _make_gather(total)(idx2d, table128)
    return out.reshape(batch, hist, d_model)


# speedup vs baseline: 1.1412x; 1.1412x over previous
import functools

import jax
import jax.numpy as jnp
from jax import lax
from jax.experimental import pallas as pl
from jax.experimental.pallas import tpu as pltpu
from jax.experimental.pallas import tpu_sc as plsc

NUM_CORES = 2
NUM_SUBCORES = 16
NUM_WORKERS = NUM_CORES * NUM_SUBCORES
LANES = 128
PACK = 4
D_MODEL = 32
IDX_W = 128
STREAMS_PER_CHUNK = 8
CHUNK = IDX_W * STREAMS_PER_CHUNK
GROUPS = IDX_W // 16


def _make_gather(total):
    assert total % (NUM_WORKERS * CHUNK) == 0
    per_worker = total // NUM_WORKERS
    n_chunks = per_worker // CHUNK
    idx_rows_per_worker = per_worker // IDX_W

    mesh = plsc.VectorSubcoreMesh(core_axis_name="c", subcore_axis_name="s")

    @functools.partial(
        pl.kernel,
        out_type=jax.ShapeDtypeStruct((total * D_MODEL,), jnp.float32),
        mesh=mesh,
        compiler_params=pltpu.CompilerParams(needs_layout_passes=False),
        scratch_types=[
            pltpu.VMEM((STREAMS_PER_CHUNK, IDX_W), jnp.int32),
            pltpu.VMEM((STREAMS_PER_CHUNK, IDX_W), jnp.int32),
            pltpu.VMEM((IDX_W, LANES), jnp.float32),
            pltpu.VMEM((IDX_W, LANES), jnp.float32),
            pltpu.VMEM((CHUNK * D_MODEL,), jnp.float32),
            pltpu.SemaphoreType.DMA,
            pltpu.SemaphoreType.DMA,
        ],
    )
    def gather_kernel(idx_hbm, table_hbm, out_hbm, idx_v, q_v, wide0, wide1,
                      comp_v, sem0, sem1):
        wid = lax.axis_index("s") * NUM_CORES + lax.axis_index("c")
        iota = lax.iota(jnp.int32, 16)

        def fire(j, w, sem):
            pltpu.async_copy(table_hbm.at[q_v.at[j]], w, sem)

        def wait(j, w, sem):
            pltpu.make_async_copy(table_hbm.at[q_v.at[j]], w, sem).wait()

        def extract(j, w):
            base_flat = j * (IDX_W * D_MODEL)

            @plsc.parallel_loop(0, GROUPS, unroll=2)
            def g_body(g):
                idx16 = idx_v[j, pl.ds(g * 16, 16)]
                rb = (idx16 & 3) * D_MODEL
                rowv = g * 16 + iota
                dst0 = base_flat + g * (16 * D_MODEL) + iota * D_MODEL
                for c in range(D_MODEL):
                    val = plsc.load_gather(w, [rowv, rb + c])
                    plsc.store_scatter(comp_v, [dst0 + c], val)

        def chunk_body(i, carry):
            row0 = wid * idx_rows_per_worker + i * STREAMS_PER_CHUNK
            pltpu.sync_copy(
                idx_hbm.at[pl.ds(row0, STREAMS_PER_CHUNK)], idx_v
            )
            for j in range(STREAMS_PER_CHUNK):
                for k in range(GROUPS):
                    sl = pl.ds(k * 16, 16)
                    q_v[j, sl] = idx_v[j, sl] >> 2
            fire(0, wide0, sem0)
            fire(1, wide1, sem1)
            for j in range(STREAMS_PER_CHUNK):
                w = wide0 if j % 2 == 0 else wide1
                sem = sem0 if j % 2 == 0 else sem1
                wait(j, w, sem)
                extract(j, w)
                if j + 2 < STREAMS_PER_CHUNK:
                    fire(j + 2, w, sem)
            out_base = (wid * per_worker + i * CHUNK) * D_MODEL
            pltpu.sync_copy(
                comp_v, out_hbm.at[pl.ds(out_base, CHUNK * D_MODEL)]
            )
            return carry

        lax.fori_loop(0, n_chunks, chunk_body, 0)

    return gather_kernel


def kernel(inputs, embedding_matrix):
    batch, hist = inputs.shape
    total = batch * hist
    vocab, d_model = embedding_matrix.shape
    idx2d = inputs.reshape(total // IDX_W, IDX_W).astype(jnp.int32)
    table128 = embedding_matrix.reshape(vocab * d_model // LANES, LANES)
    out = _make_gather(total)(idx2d, table128)
    return out.reshape(batch, hist, d_model)


# speedup vs baseline: 1.2146x; 1.0644x over previous
import functools

import jax
import jax.numpy as jnp
from jax import lax
from jax.experimental import pallas as pl
from jax.experimental.pallas import tpu as pltpu
from jax.experimental.pallas import tpu_sc as plsc

NUM_CORES = 2
NUM_SUBCORES = 16
NUM_WORKERS = NUM_CORES * NUM_SUBCORES
LANES = 128
PACK = 4
D_MODEL = 32
IDX_W = 128
STREAMS_PER_CHUNK = 8
CHUNK = IDX_W * STREAMS_PER_CHUNK
GROUPS = IDX_W // 16


def _make_gather(total):
    assert total % (NUM_WORKERS * CHUNK) == 0
    per_worker = total // NUM_WORKERS
    n_chunks = per_worker // CHUNK
    idx_rows_per_worker = per_worker // IDX_W

    mesh = plsc.VectorSubcoreMesh(core_axis_name="c", subcore_axis_name="s")

    @functools.partial(
        pl.kernel,
        out_type=jax.ShapeDtypeStruct((total * D_MODEL,), jnp.float32),
        mesh=mesh,
        compiler_params=pltpu.CompilerParams(needs_layout_passes=False),
        scratch_types=[
            pltpu.VMEM((STREAMS_PER_CHUNK, IDX_W), jnp.int32),
            pltpu.VMEM((STREAMS_PER_CHUNK, IDX_W), jnp.int32),
            pltpu.VMEM((IDX_W, LANES), jnp.float32),
            pltpu.VMEM((IDX_W, LANES), jnp.float32),
            pltpu.VMEM((IDX_W, LANES), jnp.float32),
            pltpu.VMEM((IDX_W, LANES), jnp.float32),
            pltpu.VMEM((CHUNK * D_MODEL,), jnp.float32),
            pltpu.SemaphoreType.DMA,
            pltpu.SemaphoreType.DMA,
            pltpu.SemaphoreType.DMA,
            pltpu.SemaphoreType.DMA,
        ],
    )
    def gather_kernel(idx_hbm, table_hbm, out_hbm, idx_v, q_v, wide0, wide1,
                      wide2, wide3, comp_v, sem0, sem1, sem2, sem3):
        wid = lax.axis_index("s") * NUM_CORES + lax.axis_index("c")
        iota = lax.iota(jnp.int32, 16)

        def fire(j, w, sem):
            pltpu.async_copy(table_hbm.at[q_v.at[j]], w, sem)

        def wait(j, w, sem):
            pltpu.make_async_copy(table_hbm.at[q_v.at[j]], w, sem).wait()

        def extract(j, w):
            base_flat = j * (IDX_W * D_MODEL)

            @plsc.parallel_loop(0, GROUPS, unroll=4)
            def g_body(g):
                idx16 = idx_v[j, pl.ds(g * 16, 16)]
                rb = (idx16 & 3) * D_MODEL
                rowv = g * 16 + iota
                dst0 = base_flat + g * (16 * D_MODEL) + iota * D_MODEL
                for c in range(D_MODEL):
                    val = plsc.load_gather(w, [rowv, rb + c])
                    plsc.store_scatter(comp_v, [dst0 + c], val)

        def chunk_body(i, carry):
            row0 = wid * idx_rows_per_worker + i * STREAMS_PER_CHUNK
            pltpu.sync_copy(
                idx_hbm.at[pl.ds(row0, STREAMS_PER_CHUNK)], idx_v
            )
            for j in range(STREAMS_PER_CHUNK):
                for k in range(GROUPS):
                    sl = pl.ds(k * 16, 16)
                    q_v[j, sl] = idx_v[j, sl] >> 2
            wides = (wide0, wide1, wide2, wide3)
            sems = (sem0, sem1, sem2, sem3)
            for j in range(4):
                fire(j, wides[j], sems[j])
            for j in range(STREAMS_PER_CHUNK):
                w = wides[j % 4]
                sem = sems[j % 4]
                wait(j, w, sem)
                extract(j, w)
                if j + 4 < STREAMS_PER_CHUNK:
                    fire(j + 4, w, sem)
            out_base = (wid * per_worker + i * CHUNK) * D_MODEL
            pltpu.sync_copy(
                comp_v, out_hbm.at[pl.ds(out_base, CHUNK * D_MODEL)]
            )
            return carry

        lax.fori_loop(0, n_chunks, chunk_body, 0)

    return gather_kernel


def kernel(inputs, embedding_matrix):
    batch, hist = inputs.shape
    total = batch * hist
    vocab, d_model = embedding_matrix.shape
    idx2d = inputs.reshape(total // IDX_W, IDX_W).astype(jnp.int32)
    table128 = embedding_matrix.reshape(vocab * d_model // LANES, LANES)
    out = _make_gather(total)(idx2d, table128)
    return out.reshape(batch, hist, d_model)
